# baseline (speedup 1.0000x reference)
import jax
import jax.numpy as jnp
from jax import lax
from jax.experimental import pallas as pl
from jax.experimental.pallas import tpu as pltpu
from jax.experimental.pallas import tpu_sc as plsc

D_MODEL = 64
SEQ = 200
HSEQ = SEQ // 2
NC, NS = 2, 16
NW = NC * NS
CHR = 128
LANES = 16
VPR = D_MODEL // LANES


def _emb_body(x_hbm, table_hbm, pos_hbm, out_hbm,
              idx_v, pos_v, gbuf, obuf,
              gsem0, gsem1, osem0, osem1):
    nch = x_hbm.shape[0] // NW
    pairs_w = nch * CHR // 2
    wid = lax.axis_index("s") * NC + lax.axis_index("c")

    pltpu.sync_copy(x_hbm.at[pl.ds(wid * nch, nch)], idx_v)
    pltpu.sync_copy(pos_hbm, pos_v)

    gsems = (gsem0, gsem1)
    osems = (osem0, osem1)

    def gather_copy(c, buf):
        return pltpu.make_async_copy(
            table_hbm.at[idx_v.at[c]], gbuf.at[buf], gsems[buf])

    def out_copy(c, buf):
        pair0 = wid * pairs_w + c * (CHR // 2)
        return pltpu.make_async_copy(
            obuf.at[buf], out_hbm.at[pl.ds(pair0, CHR // 2)], osems[buf])

    gather_copy(0, 0).start()
    gather_copy(1, 1).start()

    def chunk(t, b):
        c = 2 * t + b
        gather_copy(c, b).wait()
        @pl.when(t > 0)
        def _():
            out_copy(c - 2, b).wait()

        e0 = lax.rem(c * (CHR // 2), HSEQ)

        def pack_rows(fr, s):
            for j in range(VPR):
                sl = pl.ds(j * LANES, LANES)
                sh = pl.ds(D_MODEL + j * LANES, LANES)
                obuf[b, fr, sl] = gbuf[b, 2 * fr, sl] + pos_v[s, sl]
                obuf[b, fr, sh] = gbuf[b, 2 * fr + 1, sl] + pos_v[s + HSEQ, sl]
            return lax.select(s == HSEQ - 1, 0, s + 1)

        lax.fori_loop(0, CHR // 2, pack_rows, e0)

        @pl.when(c + 2 < nch)
        def _():
            gather_copy(c + 2, b).start()

        out_copy(c, b).start()

    def step(t, _):
        chunk(t, 0)
        chunk(t, 1)
        return 0

    lax.fori_loop(0, nch // 2, step, 0)

    for b in range(2):
        out_copy(nch - 2 + b, b).wait()


def kernel(x, token_table, pos_table):
    B, S = x.shape
    total = B * S
    x_perm = jnp.stack(
        [x[:, :HSEQ], x[:, HSEQ:]], axis=-1).astype(jnp.int32)
    x_lin = x_perm.reshape(total // 128, 128)

    mesh = plsc.VectorSubcoreMesh(core_axis_name="c", subcore_axis_name="s")
    packed = pl.kernel(
        _emb_body,
        out_type=jax.ShapeDtypeStruct((total // 2, 2 * D_MODEL), jnp.float32),
        mesh=mesh,
        compiler_params=pltpu.CompilerParams(use_tc_tiling_on_sc=False),
        scratch_types=[
            pltpu.VMEM((total // (NW * CHR), CHR), jnp.int32),
            pltpu.VMEM((SEQ, D_MODEL), jnp.float32),
            pltpu.VMEM((2, CHR, D_MODEL), jnp.float32),
            pltpu.VMEM((2, CHR // 2, 2 * D_MODEL), jnp.float32),
            pltpu.SemaphoreType.DMA,
            pltpu.SemaphoreType.DMA,
            pltpu.SemaphoreType.DMA,
            pltpu.SemaphoreType.DMA,
        ],
    )(x_lin, token_table, pos_table)
    p3 = packed.reshape(B, HSEQ, 2 * D_MODEL)
    return jnp.concatenate([p3[:, :, :D_MODEL], p3[:, :, D_MODEL:]], axis=1)


# speedup vs baseline: 1.5967x; 1.5967x over previous
import jax
import jax.numpy as jnp
from jax import lax
from jax.experimental import pallas as pl
from jax.experimental.pallas import tpu as pltpu
from jax.experimental.pallas import tpu_sc as plsc

D_MODEL = 64
SEQ = 200
HSEQ = SEQ // 2
NC, NS = 2, 16
NW = NC * NS
LANES = 16
VPR = D_MODEL // LANES
SEQ_W = 128
PAIRS_W = SEQ_W // 2
NCH = 2 * PAIRS_W
EPI_BP = 16


def _emb_body(x_hbm, table_hbm, pos_hbm, out_hbm,
              idx_v, pos_v, gbuf, obuf,
              gsem0, gsem1, osem0, osem1):
    wid = lax.axis_index("s") * NC + lax.axis_index("c")

    pltpu.sync_copy(x_hbm.at[wid], idx_v)
    pltpu.sync_copy(pos_hbm, pos_v)

    gsems = (gsem0, gsem1)
    osems = (osem0, osem1)

    def gather_copies(c, buf):
        r = 2 * c - buf
        return (
            pltpu.make_async_copy(
                table_hbm.at[idx_v.at[r]],
                gbuf.at[buf, pl.ds(0, HSEQ)], gsems[buf]),
            pltpu.make_async_copy(
                table_hbm.at[idx_v.at[r + 2]],
                gbuf.at[buf, pl.ds(HSEQ, HSEQ)], gsems[buf]),
        )

    def out_copy(c, buf):
        row0 = (wid * NCH + c) * HSEQ
        return pltpu.make_async_copy(
            obuf.at[buf], out_hbm.at[pl.ds(row0, HSEQ)], osems[buf])

    for ca, cb in ((0, 0), (1, 1)):
        for cp in gather_copies(ca, cb):
            cp.start()

    def chunk(t, b):
        c = 2 * t + b
        for cp in gather_copies(c, b):
            cp.wait()
        @pl.when(t > 0)
        def _():
            out_copy(c - 2, b).wait()

        def pack_rows(r, _):
            for u in range(2):
                fr = 2 * r + u
                for j in range(VPR):
                    sl = pl.ds(j * LANES, LANES)
                    sh = pl.ds(D_MODEL + j * LANES, LANES)
                    pv = pos_v[b * HSEQ + fr, sl]
                    obuf[b, fr, sl] = gbuf[b, fr, sl] + pv
                    obuf[b, fr, sh] = gbuf[b, HSEQ + fr, sl] + pv
            return 0

        lax.fori_loop(0, HSEQ // 2, pack_rows, 0)

        @pl.when(c + 2 < NCH)
        def _():
            for cp in gather_copies(c + 2, b):
                cp.start()

        out_copy(c, b).start()

    def step(t, _):
        chunk(t, 0)
        chunk(t, 1)
        return 0

    lax.fori_loop(0, NCH // 2, step, 0)

    for b in range(2):
        out_copy(NCH - 2 + b, b).wait()


def _unpack_body(pk_ref, out_ref):
    v = pk_ref[...]
    for ip in range(EPI_BP):
        blk = v[ip * SEQ:(ip + 1) * SEQ, :]
        out_ref[2 * ip] = blk[:, :D_MODEL]
        out_ref[2 * ip + 1] = blk[:, D_MODEL:]


def kernel(x, token_table, pos_table):
    B, S = x.shape
    total = B * S
    x3 = x.astype(jnp.int32).reshape(NW, total // (NW * HSEQ), HSEQ)

    mesh = plsc.VectorSubcoreMesh(core_axis_name="c", subcore_axis_name="s")
    packed = pl.kernel(
        _emb_body,
        out_type=jax.ShapeDtypeStruct((total // 2, 2 * D_MODEL), jnp.float32),
        mesh=mesh,
        compiler_params=pltpu.CompilerParams(use_tc_tiling_on_sc=False),
        scratch_types=[
            pltpu.VMEM((total // (NW * HSEQ), HSEQ), jnp.int32),
            pltpu.VMEM((SEQ, D_MODEL), jnp.float32),
            pltpu.VMEM((2, SEQ, D_MODEL), jnp.float32),
            pltpu.VMEM((2, HSEQ, 2 * D_MODEL), jnp.float32),
            pltpu.SemaphoreType.DMA,
            pltpu.SemaphoreType.DMA,
            pltpu.SemaphoreType.DMA,
            pltpu.SemaphoreType.DMA,
        ],
    )(x3, token_table, pos_table)

    grid = B // (2 * EPI_BP)
    return pl.pallas_call(
        _unpack_body,
        grid=(grid,),
        in_specs=[pl.BlockSpec((EPI_BP * SEQ, 2 * D_MODEL), lambda i: (i, 0))],
        out_specs=pl.BlockSpec((2 * EPI_BP, SEQ, D_MODEL), lambda i: (i, 0, 0)),
        out_shape=jax.ShapeDtypeStruct((B, SEQ, D_MODEL), jnp.float32),
        compiler_params=pltpu.CompilerParams(
            dimension_semantics=("arbitrary",)),
    )(packed)
